# baseline (device time: 23218 ns/iter reference)
import os

import jax
import jax.numpy as jnp
from jax import lax
from jax.experimental import pallas as pl
from jax.experimental.pallas import tpu as pltpu

try:
    _MODE = open(os.path.join(os.path.dirname(__file__), "MODE")).read().strip()
except OSError:
    _MODE = "full"

N_DEV = 32
NQ = 8
NR = 4
BLK = 32
K = 1024
N_OUT = 1024


def kernel(x, w_mat):
    def body(
        x_ref,
        w_ref,
        out_ref,
        rbuf1,
        rbuf2,
        p1_send,
        p1_recv,
        p2_send,
        p2_recv,
        loc_sems,
    ):
        me = lax.axis_index("i")
        q_me = lax.div(me, NR)
        r_me = lax.rem(me, NR)

        xv = x_ref

        barrier_sem = pltpu.get_barrier_semaphore()
        for j in range(1, NR):
            pl.semaphore_signal(
                barrier_sem,
                inc=1,
                device_id=(q_me * NR + lax.rem(r_me + j, NR),),
                device_id_type=pl.DeviceIdType.MESH,
            )
        for j in range(1, NQ):
            pl.semaphore_signal(
                barrier_sem,
                inc=1,
                device_id=(lax.rem(q_me + j, NQ) * NR + r_me,),
                device_id_type=pl.DeviceIdType.MESH,
            )
        copy1 = pltpu.make_async_copy(
            xv.at[:, r_me], rbuf1.at[r_me], loc_sems.at[0]
        )
        copy1.start()
        pl.semaphore_wait(barrier_sem, NR - 1 + NQ - 1)

        for j in range(1, NR):
            rp = lax.rem(r_me + j, NR)
            rdma = pltpu.make_async_remote_copy(
                src_ref=xv.at[:, rp],
                dst_ref=rbuf1.at[r_me],
                send_sem=p1_send.at[j],
                recv_sem=p1_recv.at[j],
                device_id=(q_me * NR + rp,),
                device_id_type=pl.DeviceIdType.MESH,
            )
            rdma.start()

        for j in range(1, NR):
            sm = lax.rem(r_me + NR - j, NR)
            recv = pltpu.make_async_remote_copy(
                src_ref=xv.at[:, sm],
                dst_ref=rbuf1.at[sm],
                send_sem=p1_send.at[j],
                recv_sem=p1_recv.at[j],
                device_id=(q_me * NR + sm,),
                device_id_type=pl.DeviceIdType.MESH,
            )
            recv.wait_recv()
        copy1.wait()

        copy2 = pltpu.make_async_copy(
            rbuf1.at[:, q_me], rbuf2.at[q_me], loc_sems.at[1]
        )
        copy2.start()

        for j in range(1, NQ):
            kt = lax.rem(q_me + j, NQ)
            rdma = pltpu.make_async_remote_copy(
                src_ref=rbuf1.at[:, kt],
                dst_ref=rbuf2.at[q_me],
                send_sem=p2_send.at[j],
                recv_sem=p2_recv.at[j],
                device_id=(kt * NR + r_me,),
                device_id_type=pl.DeviceIdType.MESH,
            )
            rdma.start()

        for j in range(1, NQ):
            sq = lax.rem(q_me + NQ - j, NQ)
            recv = pltpu.make_async_remote_copy(
                src_ref=rbuf1.at[:, sq],
                dst_ref=rbuf2.at[sq],
                send_sem=p2_send.at[j],
                recv_sem=p2_recv.at[j],
                device_id=(sq * NR + r_me,),
                device_id_type=pl.DeviceIdType.MESH,
            )
            recv.wait_recv()
        copy2.wait()

        x3 = rbuf2[...].reshape(N_DEV, BLK, BLK)
        xr = jnp.transpose(x3, (1, 0, 2)).reshape(BLK, K)
        out_ref[...] = jnp.maximum(
            jnp.dot(xr, w_ref[...], preferred_element_type=jnp.float32), 0.0
        )

        for j in range(1, NR):
            rp = lax.rem(r_me + j, NR)
            send = pltpu.make_async_remote_copy(
                src_ref=xv.at[:, rp],
                dst_ref=rbuf1.at[r_me],
                send_sem=p1_send.at[j],
                recv_sem=p1_recv.at[j],
                device_id=(q_me * NR + rp,),
                device_id_type=pl.DeviceIdType.MESH,
            )
            send.wait_send()
        for j in range(1, NQ):
            kt = lax.rem(q_me + j, NQ)
            send = pltpu.make_async_remote_copy(
                src_ref=rbuf1.at[:, kt],
                dst_ref=rbuf2.at[q_me],
                send_sem=p2_send.at[j],
                recv_sem=p2_recv.at[j],
                device_id=(kt * NR + r_me,),
                device_id_type=pl.DeviceIdType.MESH,
            )
            send.wait_send()

    return pl.pallas_call(
        body,
        out_shape=jax.ShapeDtypeStruct((BLK, N_OUT), jnp.float32),
        in_specs=[
            pl.BlockSpec(memory_space=pltpu.VMEM),
            pl.BlockSpec(memory_space=pltpu.VMEM),
        ],
        out_specs=pl.BlockSpec(memory_space=pltpu.VMEM),
        scratch_shapes=[
            pltpu.VMEM((NR, NQ, BLK, BLK), jnp.float32),
            pltpu.VMEM((NQ, NR, BLK, BLK), jnp.float32),
            pltpu.SemaphoreType.DMA((NR,)),
            pltpu.SemaphoreType.DMA((NR,)),
            pltpu.SemaphoreType.DMA((NQ,)),
            pltpu.SemaphoreType.DMA((NQ,)),
            pltpu.SemaphoreType.DMA((2,)),
        ],
        compiler_params=pltpu.CompilerParams(collective_id=0),
    )(x.reshape(NQ, NR, BLK, BLK), w_mat)


# device time: 21388 ns/iter; 1.0856x vs baseline; 1.0856x over previous
import os

import jax
import jax.numpy as jnp
from jax import lax
from jax.experimental import pallas as pl
from jax.experimental.pallas import tpu as pltpu

try:
    _MODE = open(os.path.join(os.path.dirname(__file__), "MODE")).read().strip()
except OSError:
    _MODE = "full"

N_DEV = 32
NC = 4
NK = 8
BLK = 32
K = 1024
N_OUT = 1024


def kernel(x, w_mat):
    def body(
        x_ref,
        w_ref,
        out_ref,
        rbuf1,
        rs_ref,
        rbuf2,
        p1_send,
        p1_recv,
        p2_send,
        p2_recv,
        loc_sems,
    ):
        me = lax.axis_index("i")
        c_me = lax.div(me, NK)
        k_me = lax.rem(me, NK)

        barrier_sem = pltpu.get_barrier_semaphore()
        for j in range(1, NC):
            pl.semaphore_signal(
                barrier_sem,
                inc=1,
                device_id=(lax.rem(c_me + j, NC) * NK + k_me,),
                device_id_type=pl.DeviceIdType.MESH,
            )
        for j in range(1, NK):
            pl.semaphore_signal(
                barrier_sem,
                inc=1,
                device_id=(c_me * NK + lax.rem(k_me + j, NK),),
                device_id_type=pl.DeviceIdType.MESH,
            )
        copy1 = pltpu.make_async_copy(
            x_ref.at[c_me], rbuf1.at[c_me], loc_sems.at[0]
        )
        copy1.start()
        pl.semaphore_wait(barrier_sem, NC - 1 + NK - 1)

        if _MODE == "v2bar":
            copy1.wait()
            out_ref[...] = jnp.zeros((BLK, N_OUT), jnp.float32)
            out_ref[:, 0:BLK] = rbuf1[0, 0]
            return

        for j in range(1, NC):
            cp = lax.rem(c_me + j, NC)
            rdma = pltpu.make_async_remote_copy(
                src_ref=x_ref.at[cp],
                dst_ref=rbuf1.at[c_me],
                send_sem=p1_send.at[j],
                recv_sem=p1_recv.at[j],
                device_id=(cp * NK + k_me,),
                device_id_type=pl.DeviceIdType.MESH,
            )
            rdma.start()
        for j in range(1, NC):
            cs = lax.rem(c_me + NC - j, NC)
            recv = pltpu.make_async_remote_copy(
                src_ref=x_ref.at[cs],
                dst_ref=rbuf1.at[cs],
                send_sem=p1_send.at[j],
                recv_sem=p1_recv.at[j],
                device_id=(cs * NK + k_me,),
                device_id_type=pl.DeviceIdType.MESH,
            )
            recv.wait_recv()
        copy1.wait()

        if _MODE == "v2p1":
            for j in range(1, NC):
                cp = lax.rem(c_me + j, NC)
                send = pltpu.make_async_remote_copy(
                    src_ref=x_ref.at[cp],
                    dst_ref=rbuf1.at[c_me],
                    send_sem=p1_send.at[j],
                    recv_sem=p1_recv.at[j],
                    device_id=(cp * NK + k_me,),
                    device_id_type=pl.DeviceIdType.MESH,
                )
                send.wait_send()
            out_ref[...] = jnp.zeros((BLK, N_OUT), jnp.float32)
            out_ref[:, 0:BLK] = rbuf1[0, 0]
            return

        rs_ref[...] = jnp.swapaxes(rbuf1[...], 0, 1)

        copy2 = pltpu.make_async_copy(
            rs_ref.at[k_me], rbuf2.at[k_me], loc_sems.at[1]
        )
        copy2.start()

        for j in range(1, NK):
            kp = lax.rem(k_me + j, NK)
            rdma = pltpu.make_async_remote_copy(
                src_ref=rs_ref.at[kp],
                dst_ref=rbuf2.at[k_me],
                send_sem=p2_send.at[j],
                recv_sem=p2_recv.at[j],
                device_id=(c_me * NK + kp,),
                device_id_type=pl.DeviceIdType.MESH,
            )
            rdma.start()
        for j in range(1, NK):
            ks = lax.rem(k_me + NK - j, NK)
            recv = pltpu.make_async_remote_copy(
                src_ref=rs_ref.at[ks],
                dst_ref=rbuf2.at[ks],
                send_sem=p2_send.at[j],
                recv_sem=p2_recv.at[j],
                device_id=(c_me * NK + ks,),
                device_id_type=pl.DeviceIdType.MESH,
            )
            recv.wait_recv()
        copy2.wait()

        if _MODE == "v2p2":
            out_ref[...] = jnp.zeros((BLK, N_OUT), jnp.float32)
            out_ref[:, 0:BLK] = rbuf2[0, 0]
        else:
            xr = jnp.transpose(rbuf2[...], (2, 1, 0, 3)).reshape(BLK, K)
            out_ref[...] = jnp.maximum(
                jnp.dot(xr, w_ref[...], preferred_element_type=jnp.float32),
                0.0,
            )

        for j in range(1, NC):
            cp = lax.rem(c_me + j, NC)
            send = pltpu.make_async_remote_copy(
                src_ref=x_ref.at[cp],
                dst_ref=rbuf1.at[c_me],
                send_sem=p1_send.at[j],
                recv_sem=p1_recv.at[j],
                device_id=(cp * NK + k_me,),
                device_id_type=pl.DeviceIdType.MESH,
            )
            send.wait_send()
        for j in range(1, NK):
            kp = lax.rem(k_me + j, NK)
            send = pltpu.make_async_remote_copy(
                src_ref=rs_ref.at[kp],
                dst_ref=rbuf2.at[k_me],
                send_sem=p2_send.at[j],
                recv_sem=p2_recv.at[j],
                device_id=(c_me * NK + kp,),
                device_id_type=pl.DeviceIdType.MESH,
            )
            send.wait_send()

    return pl.pallas_call(
        body,
        out_shape=jax.ShapeDtypeStruct((BLK, N_OUT), jnp.float32),
        in_specs=[
            pl.BlockSpec(memory_space=pltpu.VMEM),
            pl.BlockSpec(memory_space=pltpu.VMEM),
        ],
        out_specs=pl.BlockSpec(memory_space=pltpu.VMEM),
        scratch_shapes=[
            pltpu.VMEM((NC, NK, BLK, BLK), jnp.float32),
            pltpu.VMEM((NK, NC, BLK, BLK), jnp.float32),
            pltpu.VMEM((NK, NC, BLK, BLK), jnp.float32),
            pltpu.SemaphoreType.DMA((NC,)),
            pltpu.SemaphoreType.DMA((NC,)),
            pltpu.SemaphoreType.DMA((NK,)),
            pltpu.SemaphoreType.DMA((NK,)),
            pltpu.SemaphoreType.DMA((2,)),
        ],
        compiler_params=pltpu.CompilerParams(collective_id=0),
    )(x.reshape(NC, NK, BLK, BLK), w_mat)
